# Initial kernel scaffold; baseline (speedup 1.0000x reference)
import jax
import jax.numpy as jnp
from jax.experimental import pallas as pl


def kernel(encoded_input, retrieval_values, retrieval_scores, mention_batch_positions, mention_start_positions, mention_end_positions, mention_mask, deterministic, W, b, ln_scale, ln_bias):
    raise NotImplementedError("write your pallas kernel here")


# speedup vs baseline: 1.1332x; 1.1332x over previous
import functools

import jax
import jax.numpy as jnp
from jax import lax
from jax.experimental import pallas as pl
from jax.experimental.pallas import tpu as pltpu
from jax.experimental.pallas import tpu_sc as plsc

_EPS = 1e-06
_NC = 2
_NS = 16


def _proj_body(scores_ref, values_ref, mask_ref, w_ref, b_ref, out_ref):
    s = scores_ref[...]
    v = values_ref[...]
    weighted = jnp.sum(s[:, :, None] * v, axis=1)
    proj = jnp.dot(weighted, w_ref[...],
                   preferred_element_type=jnp.float32)
    out_ref[...] = (proj + b_ref[...]) * mask_ref[...]


def _project(scores, values, mask, W, b):
    M, K, R = values.shape
    D = W.shape[1]
    BM = 256
    grid = (M // BM,)
    return pl.pallas_call(
        _proj_body,
        grid=grid,
        in_specs=[
            pl.BlockSpec((BM, K), lambda i: (i, 0)),
            pl.BlockSpec((BM, K, R), lambda i: (i, 0, 0)),
            pl.BlockSpec((BM, 1), lambda i: (i, 0)),
            pl.BlockSpec((R, D), lambda i: (0, 0)),
            pl.BlockSpec((1, D), lambda i: (0, 0)),
        ],
        out_specs=pl.BlockSpec((BM, D), lambda i: (i, 0)),
        out_shape=jax.ShapeDtypeStruct((M, D), jnp.float32),
    )(scores, values, mask.reshape(M, 1), W, b.reshape(1, D))


def _scatter_add_sc(enc2d, proj, rows):
    BT, D = enc2d.shape
    M = proj.shape[0]
    CCH = 128
    cols_per_core = D // _NC
    n_chunks = cols_per_core // CCH
    m_per_tile = M // _NS
    r_per_tile = BT // _NS

    mesh = plsc.VectorSubcoreMesh(core_axis_name="c", subcore_axis_name="s")

    @functools.partial(
        pl.kernel,
        out_type=jax.ShapeDtypeStruct((BT, D), jnp.float32),
        mesh=mesh,
        scratch_types=[
            pltpu.VMEM_SHARED((BT, CCH), jnp.float32),
            pltpu.VMEM((m_per_tile, CCH), jnp.float32),
            pltpu.VMEM((m_per_tile,), jnp.int32),
        ],
    )
    def k(enc_hbm, proj_hbm, rows_hbm, out_hbm, slab, pv, idx_v):
        c = lax.axis_index("c")
        s = lax.axis_index("s")
        m0 = pl.multiple_of(s * m_per_tile, m_per_tile)
        r0 = pl.multiple_of(s * r_per_tile, r_per_tile)
        pltpu.sync_copy(rows_hbm.at[pl.ds(m0, m_per_tile)], idx_v)
        for ch in range(n_chunks):
            c0 = pl.multiple_of(c * cols_per_core + ch * CCH, CCH)
            pltpu.sync_copy(
                enc_hbm.at[pl.ds(r0, r_per_tile), pl.ds(c0, CCH)],
                slab.at[pl.ds(r0, r_per_tile)],
            )
            pltpu.sync_copy(
                proj_hbm.at[pl.ds(m0, m_per_tile), pl.ds(c0, CCH)], pv)
            plsc.subcore_barrier()
            pltpu.sync_copy(pv, slab.at[idx_v], add=True)
            plsc.subcore_barrier()
            pltpu.sync_copy(
                slab.at[pl.ds(r0, r_per_tile)],
                out_hbm.at[pl.ds(r0, r_per_tile), pl.ds(c0, CCH)],
            )

    return k(enc2d, proj, rows)


def _ln_body(x_ref, g_ref, b_ref, o_ref):
    x = x_ref[...]
    mean = jnp.mean(x, axis=-1, keepdims=True)
    xc = x - mean
    var = jnp.mean(xc * xc, axis=-1, keepdims=True)
    o_ref[...] = xc * lax.rsqrt(var + _EPS) * g_ref[...] + b_ref[...]


def _layernorm(x, g, b):
    BT, D = x.shape
    BR = 512
    return pl.pallas_call(
        _ln_body,
        grid=(BT // BR,),
        in_specs=[
            pl.BlockSpec((BR, D), lambda i: (i, 0)),
            pl.BlockSpec((1, D), lambda i: (0, 0)),
            pl.BlockSpec((1, D), lambda i: (0, 0)),
        ],
        out_specs=pl.BlockSpec((BR, D), lambda i: (i, 0)),
        out_shape=jax.ShapeDtypeStruct((BT, D), jnp.float32),
    )(x, g.reshape(1, D), b.reshape(1, D))


def kernel(encoded_input, retrieval_values, retrieval_scores,
           mention_batch_positions, mention_start_positions,
           mention_end_positions, mention_mask, deterministic,
           W, b, ln_scale, ln_bias):
    B, T, D = encoded_input.shape
    rows = (mention_batch_positions.astype(jnp.int32) * T
            + mention_start_positions.astype(jnp.int32))
    proj = _project(retrieval_scores, retrieval_values, mention_mask, W, b)
    enc2d = encoded_input.reshape(B * T, D)
    updated = _scatter_add_sc(enc2d, proj, rows)
    out = _layernorm(updated, ln_scale, ln_bias)
    return out.reshape(B, T, D)
